# baseline (speedup 1.0000x reference)
import jax
import jax.numpy as jnp
from jax import lax
from jax.experimental import pallas as pl
from jax.experimental.pallas import tpu as pltpu
from jax.experimental.pallas import tpu_sc as plsc

N_NODES = 10000
N_EDGES = 320000
D = 128

NC = 2
NS = 16
NW = NC * NS
F = D // NW
NP = 10112
EB = 8192
NB = 40
E_PAD = NB * EB


def _spmm_body(xt_hbm, edges_hbm, norms_hbm, out_hbm, xt_v, acc_v, eb_v, nb_v):
    c = lax.axis_index("c")
    s = lax.axis_index("s")
    wid = s * NC + c
    fbase = wid * (F * NP)

    pltpu.sync_copy(xt_hbm.at[pl.ds(fbase, F * NP)], xt_v)

    zeros16 = jnp.zeros((16,), jnp.float32)

    def _zero(i, _):
        for f in range(F):
            acc_v[pl.ds(f * NP + i * 16, 16)] = zeros16
        return 0

    lax.fori_loop(0, NP // 16, _zero, 0)

    def _block(b, _):
        pltpu.sync_copy(edges_hbm.at[b], eb_v)
        pltpu.sync_copy(norms_hbm.at[b], nb_v)

        def _grp(i, _):
            sl = pl.ds(i * 16, 16)
            srcs = eb_v[0, sl]
            dsts = eb_v[1, sl]
            norms = nb_v[sl]
            for f in range(F):
                off = f * NP
                vals = plsc.load_gather(xt_v, [srcs + off]) * norms
                plsc.addupdate_scatter(acc_v, [dsts + off], vals)
            return 0

        lax.fori_loop(0, EB // 16, _grp, 0)
        return 0

    lax.fori_loop(0, NB, _block, 0)

    pltpu.sync_copy(acc_v, out_hbm.at[pl.ds(fbase, F * NP)])


_spmm_t = pl.kernel(
    _spmm_body,
    out_type=jax.ShapeDtypeStruct((D * NP,), jnp.float32),
    mesh=plsc.VectorSubcoreMesh(core_axis_name="c", subcore_axis_name="s"),
    compiler_params=pltpu.CompilerParams(needs_layout_passes=False),
    scratch_types=[
        pltpu.VMEM((F * NP,), jnp.float32),
        pltpu.VMEM((F * NP,), jnp.float32),
        pltpu.VMEM((4, EB), jnp.int32),
        pltpu.VMEM((EB,), jnp.float32),
    ],
)


_NBLK = NP // D


def _mm_t_body(x_ref, w_ref, o_ref):
    o_ref[...] = lax.dot_general(w_ref[...], x_ref[...],
                                 (((0,), (1,)), ((), ())),
                                 preferred_element_type=jnp.float32)


def _mm_t(x, w):
    return pl.pallas_call(
        _mm_t_body,
        grid=(_NBLK,),
        in_specs=[pl.BlockSpec((D, D), lambda i: (i, 0)),
                  pl.BlockSpec((D, D), lambda i: (0, 0))],
        out_specs=pl.BlockSpec((D, D), lambda i: (0, i)),
        out_shape=jax.ShapeDtypeStruct((D, NP), jnp.float32),
    )(x, w)


def _fuse_t_body(y_ref, b_ref, w_ref, o_ref):
    h = jnp.maximum(y_ref[...] + b_ref[...], 0.0)
    o_ref[...] = lax.dot_general(w_ref[...], h,
                                 (((0,), (0,)), ((), ())),
                                 preferred_element_type=jnp.float32)


def _fuse_t(yt, bcol, w):
    return pl.pallas_call(
        _fuse_t_body,
        grid=(_NBLK,),
        in_specs=[pl.BlockSpec((D, D), lambda i: (0, i)),
                  pl.BlockSpec((D, D), lambda i: (0, 0)),
                  pl.BlockSpec((D, D), lambda i: (0, 0))],
        out_specs=pl.BlockSpec((D, D), lambda i: (0, i)),
        out_shape=jax.ShapeDtypeStruct((D, NP), jnp.float32),
    )(yt, bcol, w)


def _final_t_body(q_ref, i_ref, b_ref, o_ref):
    o_ref[...] = lax.dot_general(q_ref[...], i_ref[...],
                                 (((0,), (0,)), ((), ())),
                                 preferred_element_type=jnp.float32) + b_ref[...]


def _final_t(qt, b):
    eye = jnp.eye(D, dtype=jnp.float32)
    return pl.pallas_call(
        _final_t_body,
        grid=(_NBLK,),
        in_specs=[pl.BlockSpec((D, D), lambda i: (0, i)),
                  pl.BlockSpec((D, D), lambda i: (0, 0)),
                  pl.BlockSpec((1, D), lambda i: (0, 0))],
        out_specs=pl.BlockSpec((D, D), lambda i: (i, 0)),
        out_shape=jax.ShapeDtypeStruct((NP, D), jnp.float32),
    )(qt, eye, b.reshape(1, D))


def kernel(feat, edge_index, norm_data, W1, b1, W2, b2):
    src = edge_index[0].astype(jnp.int32)
    dst = edge_index[1].astype(jnp.int32)
    norm = norm_data.astype(jnp.float32)

    pad = E_PAD - N_EDGES
    srcs = jnp.concatenate([src, jnp.zeros((pad,), jnp.int32)]).reshape(NB, EB)
    dsts = jnp.concatenate([dst, jnp.zeros((pad,), jnp.int32)]).reshape(NB, EB)
    norms = jnp.concatenate([norm, jnp.zeros((pad,), jnp.float32)]).reshape(NB, EB)
    edges = jnp.stack([srcs, dsts, jnp.zeros_like(srcs), jnp.zeros_like(srcs)],
                      axis=1)

    feat_p = jnp.pad(feat, ((0, NP - N_NODES), (0, 0)))
    b1c = jnp.broadcast_to(b1[:, None], (D, D))

    x1t = _mm_t(feat_p, W1).reshape(D * NP)
    y1t = _spmm_t(x1t, edges, norms).reshape(D, NP)
    x2t = _fuse_t(y1t, b1c, W2).reshape(D * NP)
    y2t = _spmm_t(x2t, edges, norms).reshape(D, NP)
    return _final_t(y2t, b2)[:N_NODES]


# speedup vs baseline: 1.0149x; 1.0149x over previous
import jax
import jax.numpy as jnp
from jax import lax
from jax.experimental import pallas as pl
from jax.experimental.pallas import tpu as pltpu
from jax.experimental.pallas import tpu_sc as plsc

N_NODES = 10000
N_EDGES = 320000
D = 128

NC = 2
NS = 16
NW = NC * NS
F = D // NW
NP = 10112
EB = 8192
NB = 40
E_PAD = NB * EB


def _spmm_body(xt_hbm, edges_hbm, norms_hbm, out_hbm, xt_v, acc_v, eb_v, nb_v):
    c = lax.axis_index("c")
    s = lax.axis_index("s")
    wid = s * NC + c
    fbase = wid * (F * NP)

    pltpu.sync_copy(xt_hbm.at[pl.ds(fbase, F * NP)], xt_v)

    zeros16 = jnp.zeros((16,), jnp.float32)

    def _zero(i, _):
        for f in range(F):
            acc_v[pl.ds(f * NP + i * 16, 16)] = zeros16
        return 0

    lax.fori_loop(0, NP // 16, _zero, 0)

    def _block(b, _):
        pltpu.sync_copy(edges_hbm.at[b], eb_v)
        pltpu.sync_copy(norms_hbm.at[b], nb_v)

        def _grp(i, _):
            for u in range(4):
                sl = pl.ds(i * 64 + u * 16, 16)
                srcs = eb_v[0, sl]
                dsts = eb_v[1, sl]
                norms = nb_v[sl]
                for f in range(F):
                    off = f * NP
                    vals = plsc.load_gather(xt_v, [srcs + off]) * norms
                    plsc.addupdate_scatter(acc_v, [dsts + off], vals)
            return 0

        lax.fori_loop(0, EB // 64, _grp, 0)
        return 0

    lax.fori_loop(0, NB, _block, 0)

    pltpu.sync_copy(acc_v, out_hbm.at[pl.ds(fbase, F * NP)])


_spmm_t = pl.kernel(
    _spmm_body,
    out_type=jax.ShapeDtypeStruct((D * NP,), jnp.float32),
    mesh=plsc.VectorSubcoreMesh(core_axis_name="c", subcore_axis_name="s"),
    compiler_params=pltpu.CompilerParams(needs_layout_passes=False),
    scratch_types=[
        pltpu.VMEM((F * NP,), jnp.float32),
        pltpu.VMEM((F * NP,), jnp.float32),
        pltpu.VMEM((4, EB), jnp.int32),
        pltpu.VMEM((EB,), jnp.float32),
    ],
)


_NBLK = NP // D


def _mm_t_body(x_ref, w_ref, o_ref):
    o_ref[...] = lax.dot_general(w_ref[...], x_ref[...],
                                 (((0,), (1,)), ((), ())),
                                 preferred_element_type=jnp.float32)


def _mm_t(x, w):
    return pl.pallas_call(
        _mm_t_body,
        grid=(_NBLK,),
        in_specs=[pl.BlockSpec((D, D), lambda i: (i, 0)),
                  pl.BlockSpec((D, D), lambda i: (0, 0))],
        out_specs=pl.BlockSpec((D, D), lambda i: (0, i)),
        out_shape=jax.ShapeDtypeStruct((D, NP), jnp.float32),
    )(x, w)


def _fuse_t_body(y_ref, b_ref, w_ref, o_ref):
    h = jnp.maximum(y_ref[...] + b_ref[...], 0.0)
    o_ref[...] = lax.dot_general(w_ref[...], h,
                                 (((0,), (0,)), ((), ())),
                                 preferred_element_type=jnp.float32)


def _fuse_t(yt, bcol, w):
    return pl.pallas_call(
        _fuse_t_body,
        grid=(_NBLK,),
        in_specs=[pl.BlockSpec((D, D), lambda i: (0, i)),
                  pl.BlockSpec((D, D), lambda i: (0, 0)),
                  pl.BlockSpec((D, D), lambda i: (0, 0))],
        out_specs=pl.BlockSpec((D, D), lambda i: (0, i)),
        out_shape=jax.ShapeDtypeStruct((D, NP), jnp.float32),
    )(yt, bcol, w)


def _final_t_body(q_ref, i_ref, b_ref, o_ref):
    o_ref[...] = lax.dot_general(q_ref[...], i_ref[...],
                                 (((0,), (0,)), ((), ())),
                                 preferred_element_type=jnp.float32) + b_ref[...]


def _final_t(qt, b):
    eye = jnp.eye(D, dtype=jnp.float32)
    return pl.pallas_call(
        _final_t_body,
        grid=(_NBLK,),
        in_specs=[pl.BlockSpec((D, D), lambda i: (0, i)),
                  pl.BlockSpec((D, D), lambda i: (0, 0)),
                  pl.BlockSpec((1, D), lambda i: (0, 0))],
        out_specs=pl.BlockSpec((D, D), lambda i: (i, 0)),
        out_shape=jax.ShapeDtypeStruct((NP, D), jnp.float32),
    )(qt, eye, b.reshape(1, D))


def kernel(feat, edge_index, norm_data, W1, b1, W2, b2):
    src = edge_index[0].astype(jnp.int32)
    dst = edge_index[1].astype(jnp.int32)
    norm = norm_data.astype(jnp.float32)

    pad = E_PAD - N_EDGES
    srcs = jnp.concatenate([src, jnp.zeros((pad,), jnp.int32)]).reshape(NB, EB)
    dsts = jnp.concatenate([dst, jnp.zeros((pad,), jnp.int32)]).reshape(NB, EB)
    norms = jnp.concatenate([norm, jnp.zeros((pad,), jnp.float32)]).reshape(NB, EB)
    edges = jnp.stack([srcs, dsts, jnp.zeros_like(srcs), jnp.zeros_like(srcs)],
                      axis=1)

    feat_p = jnp.pad(feat, ((0, NP - N_NODES), (0, 0)))
    b1c = jnp.broadcast_to(b1[:, None], (D, D))

    x1t = _mm_t(feat_p, W1).reshape(D * NP)
    y1t = _spmm_t(x1t, edges, norms).reshape(D, NP)
    x2t = _fuse_t(y1t, b1c, W2).reshape(D * NP)
    y2t = _spmm_t(x2t, edges, norms).reshape(D, NP)
    return _final_t(y2t, b2)[:N_NODES]


# speedup vs baseline: 2.1935x; 2.1613x over previous
import jax
import jax.numpy as jnp
from jax import lax
from jax.experimental import pallas as pl
from jax.experimental.pallas import tpu as pltpu
from jax.experimental.pallas import tpu_sc as plsc

N_NODES = 10000
N_EDGES = 320000
D = 128

NC = 2
NS = 16
NW = NC * NS
CH = 128
NCH = -(-N_EDGES // (NW * CH))
E_TILE = NCH * CH
E_PAD = NW * E_TILE

ROWS_MAIN = 624
ROWS_TAIL = N_NODES - NS * ROWS_MAIN


def _spmm_body(x_hbm, srcs_hbm, dsts_hbm, norms_hbm, out_hbm,
               src_v, dst_v, norm_v, rows_v, acc_sh, sem):
    c = lax.axis_index("c")
    s = lax.axis_index("s")
    wid = s * NC + c

    zeros16 = jnp.zeros((16,), jnp.float32)

    def _zero_row(r, _):
        for b in range(D // 16):
            rows_v[r, pl.ds(b * 16, 16)] = zeros16
        return 0

    lax.fori_loop(0, CH, _zero_row, 0)

    base = s * ROWS_MAIN
    for off, size in ((0, 128), (128, 128), (256, 128), (384, 128), (512, 112)):
        pltpu.sync_copy(rows_v.at[pl.ds(0, size)],
                        acc_sh.at[pl.ds(base + off, size)])

    @pl.when(s == NS - 1)
    def _():
        pltpu.sync_copy(rows_v.at[pl.ds(0, ROWS_TAIL)],
                        acc_sh.at[pl.ds(NS * ROWS_MAIN, ROWS_TAIL)])

    plsc.subcore_barrier()

    pltpu.sync_copy(srcs_hbm.at[wid], src_v)
    pltpu.sync_copy(dsts_hbm.at[wid], dst_v)
    pltpu.sync_copy(norms_hbm.at[wid], norm_v)

    def _chunk(j, _):
        pltpu.async_copy(x_hbm.at[src_v.at[j]], rows_v, sem).wait()

        def _scale_grp(g, _):
            nv16 = norm_v[j, pl.ds(g * 16, 16)]
            e0 = g * 16
            for ei in range(16):
                nv = jnp.full((16,), nv16[ei], jnp.float32)
                for b in range(D // 16):
                    sl = pl.ds(b * 16, 16)
                    rows_v[e0 + ei, sl] = rows_v[e0 + ei, sl] * nv
            return 0

        lax.fori_loop(0, CH // 16, _scale_grp, 0)
        pltpu.sync_copy(rows_v, acc_sh.at[dst_v.at[j]], add=True)
        return 0

    lax.fori_loop(0, NCH, _chunk, 0)

    plsc.subcore_barrier()

    pltpu.sync_copy(acc_sh.at[pl.ds(base, ROWS_MAIN)],
                    out_hbm.at[c, pl.ds(base, ROWS_MAIN)])

    @pl.when(s == NS - 1)
    def _():
        pltpu.sync_copy(acc_sh.at[pl.ds(NS * ROWS_MAIN, ROWS_TAIL)],
                        out_hbm.at[c, pl.ds(NS * ROWS_MAIN, ROWS_TAIL)])


_spmm = pl.kernel(
    _spmm_body,
    out_type=jax.ShapeDtypeStruct((NC, N_NODES, D), jnp.float32),
    mesh=plsc.VectorSubcoreMesh(core_axis_name="c", subcore_axis_name="s"),
    scratch_types=[
        pltpu.VMEM((NCH, CH), jnp.int32),
        pltpu.VMEM((NCH, CH), jnp.int32),
        pltpu.VMEM((NCH, CH), jnp.float32),
        pltpu.VMEM((CH, D), jnp.float32),
        pltpu.VMEM_SHARED((N_NODES, D), jnp.float32),
        pltpu.SemaphoreType.DMA,
    ],
)


_BLK = 1000


def _mm_body(x_ref, w_ref, o_ref):
    o_ref[...] = jnp.dot(x_ref[...], w_ref[...],
                         preferred_element_type=jnp.float32)


def _mm(x, w):
    n = x.shape[0]
    return pl.pallas_call(
        _mm_body,
        grid=(n // _BLK,),
        in_specs=[pl.BlockSpec((_BLK, D), lambda i: (i, 0)),
                  pl.BlockSpec((D, D), lambda i: (0, 0))],
        out_specs=pl.BlockSpec((_BLK, D), lambda i: (i, 0)),
        out_shape=jax.ShapeDtypeStruct((n, D), jnp.float32),
    )(x, w)


def _fuse_body(p_ref, b_ref, w_ref, o_ref):
    h = p_ref[0] + p_ref[1] + b_ref[...]
    h = jnp.maximum(h, 0.0)
    o_ref[...] = jnp.dot(h, w_ref[...], preferred_element_type=jnp.float32)


def _fuse_relu_mm(parts, b, w):
    n = parts.shape[1]
    return pl.pallas_call(
        _fuse_body,
        grid=(n // _BLK,),
        in_specs=[pl.BlockSpec((2, _BLK, D), lambda i: (0, i, 0)),
                  pl.BlockSpec((1, D), lambda i: (0, 0)),
                  pl.BlockSpec((D, D), lambda i: (0, 0))],
        out_specs=pl.BlockSpec((_BLK, D), lambda i: (i, 0)),
        out_shape=jax.ShapeDtypeStruct((n, D), jnp.float32),
    )(parts, b.reshape(1, D), w)


def _final_body(q_ref, b_ref, o_ref):
    o_ref[...] = q_ref[0] + q_ref[1] + b_ref[...]


def _final_add(parts, b):
    n = parts.shape[1]
    return pl.pallas_call(
        _final_body,
        grid=(n // _BLK,),
        in_specs=[pl.BlockSpec((2, _BLK, D), lambda i: (0, i, 0)),
                  pl.BlockSpec((1, D), lambda i: (0, 0))],
        out_specs=pl.BlockSpec((_BLK, D), lambda i: (i, 0)),
        out_shape=jax.ShapeDtypeStruct((n, D), jnp.float32),
    )(parts, b.reshape(1, D))


def kernel(feat, edge_index, norm_data, W1, b1, W2, b2):
    src = edge_index[0].astype(jnp.int32)
    dst = edge_index[1].astype(jnp.int32)
    norm = norm_data.astype(jnp.float32)

    pad = E_PAD - N_EDGES
    srcs = jnp.concatenate([src, jnp.zeros((pad,), jnp.int32)]).reshape(NW, NCH, CH)
    dsts = jnp.concatenate([dst, jnp.zeros((pad,), jnp.int32)]).reshape(NW, NCH, CH)
    norms = jnp.concatenate([norm, jnp.zeros((pad,), jnp.float32)]).reshape(NW, NCH, CH)

    x1 = _mm(feat, W1)
    p = _spmm(x1, srcs, dsts, norms)
    x2 = _fuse_relu_mm(p, b1, W2)
    q = _spmm(x2, srcs, dsts, norms)
    return _final_add(q, b2)
